# baseline (speedup 1.0000x reference)
import functools

import jax
import jax.numpy as jnp
from jax import lax
from jax.experimental import pallas as pl
from jax.experimental.pallas import tpu as pltpu
from jax.experimental.pallas import tpu_sc as plsc

EMB = 128
R0 = 119
RC = 288
MAX_NORM = 10.0

NC = 2
NS = 16
NW = NC * NS
GROUP = 128
GROUPS_PER_W = 25
PER_W = GROUP * GROUPS_PER_W
N_PAD = NW * PER_W

DEPTH = 2


def _renorm(w):
    norm = jnp.sqrt(jnp.sum(w * w, axis=-1, keepdims=True))
    scale = jnp.where(norm > MAX_NORM, MAX_NORM / (norm + 1e-7), 1.0)
    return w * scale


def _build_body(w0_ref, w1_ref, w2_ref, w3_ref, w4_ref, t0_ref, t1_ref):
    t0_ref[...] = _renorm(w0_ref[...])
    r1 = _renorm(w1_ref[...])
    r2 = _renorm(w2_ref[...])
    r3 = _renorm(w3_ref[...])
    r4 = _renorm(w4_ref[...])
    r34 = jnp.concatenate([r3[0:1] + r4, r3[1:2] + r4], axis=0)
    r234 = jnp.concatenate([r2[k:k + 1] + r34 for k in range(6)], axis=0)
    t1_ref[...] = jnp.concatenate(
        [r1[a:a + 1] + r234 for a in range(12)], axis=0)


def _build_tables(W0, W1, W2, W3, W4):
    return pl.pallas_call(
        _build_body,
        out_shape=[
            jax.ShapeDtypeStruct((R0, EMB), jnp.float32),
            jax.ShapeDtypeStruct((RC, EMB), jnp.float32),
        ],
    )(W0, W1, W2, W3, W4)


def _sc_lookup_body(x0h, x1h, x2h, x3h, x4h, t0h, t1h, outh,
                    xb0, xb1, xb2, xb3, xb4, t0b, t1b, rows3, osem):
    wid = lax.axis_index("s") * NC + lax.axis_index("c")
    base = wid * PER_W
    cps = [pltpu.async_copy(xh.at[pl.ds(base, PER_W)], xb, osem)
           for xh, xb in ((x0h, xb0), (x1h, xb1), (x2h, xb2),
                          (x3h, xb3), (x4h, xb4))]
    cps.append(pltpu.async_copy(t0h, t0b, osem))
    cps.append(pltpu.async_copy(t1h, t1b, osem))
    for cp in cps:
        cp.wait()

    iota16 = lax.iota(jnp.int32, 16)
    ones = jnp.full((16,), 1, jnp.int32)
    mask127 = jnp.full((16,), EMB - 1, jnp.int32)

    def wait_one_out():
        pltpu.make_async_copy(
            rows3.at[0], outh.at[pl.ds(base, GROUP)], osem).wait()

    def j_body(j, carry):
        p = lax.rem(j, DEPTH)

        @pl.when(j >= DEPTH)
        def _():
            wait_one_out()

        pv = jnp.full((16,), p, jnp.int32)

        def g_body(g, carry2):
            q = j * GROUP + g * 16
            row_a = xb0[pl.ds(q, 16)]
            row_b = (xb1[pl.ds(q, 16)] * 24
                     + xb2[pl.ds(q, 16)] * 4
                     + xb3[pl.ds(q, 16)] * 2
                     + xb4[pl.ds(q, 16)])
            nodev = g * 16 + iota16

            @plsc.parallel_loop(0, EMB, 1, unroll=8)
            def _cols(c):
                colv = jnp.full((16,), c, jnp.int32)
                va = plsc.load_gather(t0b, [row_a, colv])
                vb = plsc.load_gather(t1b, [row_b, colv])
                plsc.store_scatter(rows3, [pv, nodev, colv], va + vb)

            return carry2

        lax.fori_loop(0, GROUP // 16, g_body, 0)
        pltpu.async_copy(
            rows3.at[p], outh.at[pl.ds(base + j * GROUP, GROUP)], osem)
        return carry

    lax.fori_loop(0, GROUPS_PER_W, j_body, 0)
    for _ in range(DEPTH):
        wait_one_out()


@functools.cache
def _make_sc_lookup():
    mesh = plsc.VectorSubcoreMesh(
        core_axis_name="c", subcore_axis_name="s",
        num_cores=NC, num_subcores=NS)
    return pl.kernel(
        _sc_lookup_body,
        out_type=jax.ShapeDtypeStruct((N_PAD, EMB), jnp.float32),
        mesh=mesh,
        compiler_params=pltpu.CompilerParams(needs_layout_passes=False),
        scratch_types=[
            pltpu.VMEM((PER_W,), jnp.int32),
            pltpu.VMEM((PER_W,), jnp.int32),
            pltpu.VMEM((PER_W,), jnp.int32),
            pltpu.VMEM((PER_W,), jnp.int32),
            pltpu.VMEM((PER_W,), jnp.int32),
            pltpu.VMEM((R0, EMB), jnp.float32),
            pltpu.VMEM((RC, EMB), jnp.float32),
            pltpu.VMEM((DEPTH, GROUP, EMB), jnp.float32),
            pltpu.SemaphoreType.DMA,
        ],
    )


def kernel(x, W0, W1, W2, W3, W4):
    n = x.shape[0]
    x = x.astype(jnp.int32)
    t0, t1 = _build_tables(W0, W1, W2, W3, W4)
    xt = jnp.pad(x.T, ((0, 0), (0, N_PAD - n)))
    out = _make_sc_lookup()(xt[0], xt[1], xt[2], xt[3], xt[4], t0, t1)
    return out[:n]


# speedup vs baseline: 4.4671x; 4.4671x over previous
import functools

import jax
import jax.numpy as jnp
from jax import lax
from jax.experimental import pallas as pl
from jax.experimental.pallas import tpu as pltpu
from jax.experimental.pallas import tpu_sc as plsc

EMB = 128
R0 = 119
RC = 288
MAX_NORM = 10.0

NC = 2
NS = 16
NW = NC * NS
GROUP = 128
GROUPS_PER_W = 25
PER_W = GROUP * GROUPS_PER_W
N_PAD = NW * PER_W

DEPTH = 2


def _renorm(w):
    norm = jnp.sqrt(jnp.sum(w * w, axis=-1, keepdims=True))
    scale = jnp.where(norm > MAX_NORM, MAX_NORM / (norm + 1e-7), 1.0)
    return w * scale


def _build_body(w0_ref, w1_ref, w2_ref, w3_ref, w4_ref, t0_ref, t1_ref):
    t0_ref[...] = _renorm(w0_ref[...])
    r1 = _renorm(w1_ref[...])
    r2 = _renorm(w2_ref[...])
    r3 = _renorm(w3_ref[...])
    r4 = _renorm(w4_ref[...])
    r34 = jnp.concatenate([r3[0:1] + r4, r3[1:2] + r4], axis=0)
    r234 = jnp.concatenate([r2[k:k + 1] + r34 for k in range(6)], axis=0)
    t1_ref[...] = jnp.concatenate(
        [r1[a:a + 1] + r234 for a in range(12)], axis=0)


def _build_tables(W0, W1, W2, W3, W4):
    return pl.pallas_call(
        _build_body,
        out_shape=[
            jax.ShapeDtypeStruct((R0, EMB), jnp.float32),
            jax.ShapeDtypeStruct((RC, EMB), jnp.float32),
        ],
    )(W0, W1, W2, W3, W4)


def _sc_lookup_body(x0h, x1h, x2h, x3h, x4h, t0h, t1h, outh,
                    xb0, xb1, xb2, xb3, xb4, t0b, t1b, rows3, osem):
    wid = lax.axis_index("s") * NC + lax.axis_index("c")
    base = wid * PER_W
    cps = [pltpu.async_copy(xh.at[pl.ds(base, PER_W)], xb, osem)
           for xh, xb in ((x0h, xb0), (x1h, xb1), (x2h, xb2),
                          (x3h, xb3), (x4h, xb4))]
    cps.append(pltpu.async_copy(t0h, t0b, osem))
    cps.append(pltpu.async_copy(t1h, t1b, osem))
    for cp in cps:
        cp.wait()

    iota16 = lax.iota(jnp.int32, 16)
    ones = jnp.full((16,), 1, jnp.int32)
    mask127 = jnp.full((16,), EMB - 1, jnp.int32)

    def wait_one_out():
        pltpu.make_async_copy(
            rows3.at[0], outh.at[pl.ds(base, GROUP)], osem).wait()

    def j_body(j, carry):
        p = lax.rem(j, DEPTH)

        @pl.when(j >= DEPTH)
        def _():
            wait_one_out()

        pv = jnp.full((16,), p, jnp.int32)

        def g_body(g, carry2):
            q = j * GROUP + g * 16
            row_a = xb0[pl.ds(q, 16)]
            row_b = (xb1[pl.ds(q, 16)] * 24
                     + xb2[pl.ds(q, 16)] * 4
                     + xb3[pl.ds(q, 16)] * 2
                     + xb4[pl.ds(q, 16)])
            nodev = g * 16 + iota16

            @plsc.parallel_loop(0, EMB, 1, unroll=8)
            def _cols(c):
                cv = (iota16 + c) & mask127
                va = plsc.load_gather(t0b, [row_a, cv])
                vb = plsc.load_gather(t1b, [row_b, cv])
                plsc.store_scatter(rows3, [pv, nodev, cv], va + vb)

            return carry2

        lax.fori_loop(0, GROUP // 16, g_body, 0)
        pltpu.async_copy(
            rows3.at[p], outh.at[pl.ds(base + j * GROUP, GROUP)], osem)
        return carry

    lax.fori_loop(0, GROUPS_PER_W, j_body, 0)
    for _ in range(DEPTH):
        wait_one_out()


@functools.cache
def _make_sc_lookup():
    mesh = plsc.VectorSubcoreMesh(
        core_axis_name="c", subcore_axis_name="s",
        num_cores=NC, num_subcores=NS)
    return pl.kernel(
        _sc_lookup_body,
        out_type=jax.ShapeDtypeStruct((N_PAD, EMB), jnp.float32),
        mesh=mesh,
        compiler_params=pltpu.CompilerParams(needs_layout_passes=False),
        scratch_types=[
            pltpu.VMEM((PER_W,), jnp.int32),
            pltpu.VMEM((PER_W,), jnp.int32),
            pltpu.VMEM((PER_W,), jnp.int32),
            pltpu.VMEM((PER_W,), jnp.int32),
            pltpu.VMEM((PER_W,), jnp.int32),
            pltpu.VMEM((R0, EMB), jnp.float32),
            pltpu.VMEM((RC, EMB), jnp.float32),
            pltpu.VMEM((DEPTH, GROUP, EMB), jnp.float32),
            pltpu.SemaphoreType.DMA,
        ],
    )


def kernel(x, W0, W1, W2, W3, W4):
    n = x.shape[0]
    x = x.astype(jnp.int32)
    t0, t1 = _build_tables(W0, W1, W2, W3, W4)
    xt = jnp.pad(x.T, ((0, 0), (0, N_PAD - n)))
    out = _make_sc_lookup()(xt[0], xt[1], xt[2], xt[3], xt[4], t0, t1)
    return out[:n]


# speedup vs baseline: 6.2844x; 1.4068x over previous
import functools

import jax
import jax.numpy as jnp
from jax import lax
from jax.experimental import pallas as pl
from jax.experimental.pallas import tpu as pltpu
from jax.experimental.pallas import tpu_sc as plsc

EMB = 128
R0 = 119
RC = 288
MAX_NORM = 10.0

NC = 2
NS = 16
NW = NC * NS
GROUP = 128
GROUPS_PER_W = 25
PER_W = GROUP * GROUPS_PER_W
N_PAD = NW * PER_W

DEPTH = 2


def _renorm(w):
    norm = jnp.sqrt(jnp.sum(w * w, axis=-1, keepdims=True))
    scale = jnp.where(norm > MAX_NORM, MAX_NORM / (norm + 1e-7), 1.0)
    return w * scale


def _build_body(w0_ref, w1_ref, w2_ref, w3_ref, w4_ref, t0_ref, t1_ref):
    t0_ref[...] = _renorm(w0_ref[...])
    r1 = _renorm(w1_ref[...])
    r2 = _renorm(w2_ref[...])
    r3 = _renorm(w3_ref[...])
    r4 = _renorm(w4_ref[...])
    r34 = jnp.concatenate([r3[0:1] + r4, r3[1:2] + r4], axis=0)
    r234 = jnp.concatenate([r2[k:k + 1] + r34 for k in range(6)], axis=0)
    t1_ref[...] = jnp.concatenate(
        [r1[a:a + 1] + r234 for a in range(12)], axis=0)


def _build_tables(W0, W1, W2, W3, W4):
    return pl.pallas_call(
        _build_body,
        out_shape=[
            jax.ShapeDtypeStruct((R0, EMB), jnp.float32),
            jax.ShapeDtypeStruct((RC, EMB), jnp.float32),
        ],
    )(W0, W1, W2, W3, W4)


def _sc_lookup_body(n, x0h, x1h, x2h, x3h, x4h, t0h, t1h, outh,
                    xb0, xb1, xb2, xb3, xb4, t0b, t1b, rows3, osem):
    wid = lax.axis_index("s") * NC + lax.axis_index("c")
    last_base = n - PER_W
    step = last_base // (NW - 1)
    base = pl.multiple_of(
        jnp.where(wid == NW - 1, last_base, (wid * step) & ~7), 8)
    cps = [pltpu.async_copy(xh.at[pl.ds(base, PER_W)], xb, osem)
           for xh, xb in ((x0h, xb0), (x1h, xb1), (x2h, xb2),
                          (x3h, xb3), (x4h, xb4))]
    cps.append(pltpu.async_copy(t0h, t0b, osem))
    cps.append(pltpu.async_copy(t1h, t1b, osem))
    for cp in cps:
        cp.wait()

    iota16 = lax.iota(jnp.int32, 16)
    ones = jnp.full((16,), 1, jnp.int32)
    mask127 = jnp.full((16,), EMB - 1, jnp.int32)

    def wait_one_out():
        pltpu.make_async_copy(
            rows3.at[0], outh.at[pl.ds(base, GROUP)], osem).wait()

    def j_body(j, carry):
        p = lax.rem(j, DEPTH)

        @pl.when(j >= DEPTH)
        def _():
            wait_one_out()

        pv = jnp.full((16,), p, jnp.int32)

        def g_body(g, carry2):
            q = j * GROUP + g * 16
            row_a = xb0[pl.ds(q, 16)]
            row_b = (xb1[pl.ds(q, 16)] * 24
                     + xb2[pl.ds(q, 16)] * 4
                     + xb3[pl.ds(q, 16)] * 2
                     + xb4[pl.ds(q, 16)])
            nodev = g * 16 + iota16

            @plsc.parallel_loop(0, EMB, 1, unroll=8)
            def _cols(c):
                cv = (iota16 + c) & mask127
                va = plsc.load_gather(t0b, [row_a, cv])
                vb = plsc.load_gather(t1b, [row_b, cv])
                plsc.store_scatter(rows3, [pv, nodev, cv], va + vb)

            return carry2

        lax.fori_loop(0, GROUP // 16, g_body, 0)
        pltpu.async_copy(
            rows3.at[p], outh.at[pl.ds(base + j * GROUP, GROUP)], osem)
        return carry

    lax.fori_loop(0, GROUPS_PER_W, j_body, 0)
    for _ in range(DEPTH):
        wait_one_out()


@functools.cache
def _make_sc_lookup(n):
    mesh = plsc.VectorSubcoreMesh(
        core_axis_name="c", subcore_axis_name="s",
        num_cores=NC, num_subcores=NS)
    return pl.kernel(
        functools.partial(_sc_lookup_body, n),
        out_type=jax.ShapeDtypeStruct((n, EMB), jnp.float32),
        mesh=mesh,
        compiler_params=pltpu.CompilerParams(needs_layout_passes=False),
        scratch_types=[
            pltpu.VMEM((PER_W,), jnp.int32),
            pltpu.VMEM((PER_W,), jnp.int32),
            pltpu.VMEM((PER_W,), jnp.int32),
            pltpu.VMEM((PER_W,), jnp.int32),
            pltpu.VMEM((PER_W,), jnp.int32),
            pltpu.VMEM((R0, EMB), jnp.float32),
            pltpu.VMEM((RC, EMB), jnp.float32),
            pltpu.VMEM((DEPTH, GROUP, EMB), jnp.float32),
            pltpu.SemaphoreType.DMA,
        ],
    )


def kernel(x, W0, W1, W2, W3, W4):
    n = x.shape[0]
    x = x.astype(jnp.int32)
    t0, t1 = _build_tables(W0, W1, W2, W3, W4)
    xt = x.T
    return _make_sc_lookup(n)(xt[0], xt[1], xt[2], xt[3], xt[4], t0, t1)
